# Initial kernel scaffold; baseline (speedup 1.0000x reference)
import jax
import jax.numpy as jnp
from jax.experimental import pallas as pl


def kernel(x, w_gate, e_w1, e_w2, s_w1, s_w2):
    raise NotImplementedError("write your pallas kernel here")


# speedup vs baseline: 1.1103x; 1.1103x over previous
import functools

import jax
import jax.numpy as jnp
from jax import lax
from jax.experimental import pallas as pl
from jax.experimental.pallas import tpu as pltpu

T = 2048
D = 2048
E = 8
K = 2
H = 1408
H2 = 2 * H

B = 256
NB = (K * T) // B + E
S = K * T
S_PAD = NB * B


def _gate_body(x_ref, wg_ref, w_out_ref, e_out_ref):
    sT = lax.dot_general(wg_ref[...], x_ref[...], (((1,), (1,)), ((), ())),
                         preferred_element_type=jnp.float32)
    bt = sT.shape[1]
    m = jnp.max(sT, axis=0, keepdims=True)
    p = jnp.exp(sT - m)
    denom = jnp.sum(p, axis=0, keepdims=True)
    rows = lax.broadcasted_iota(jnp.int32, (E, bt), 0)
    p1 = jnp.max(p, axis=0, keepdims=True)
    e1 = jnp.min(jnp.where(p == p1, rows, E), axis=0, keepdims=True)
    p_m = jnp.where(rows == e1, -jnp.inf, p)
    p2 = jnp.max(p_m, axis=0, keepdims=True)
    e2 = jnp.min(jnp.where(p_m == p2, rows, E), axis=0, keepdims=True)
    w_out_ref[...] = jnp.concatenate([p1, p2], axis=0) / denom
    e_out_ref[...] = jnp.concatenate([e1, e2], axis=0)


def _gate(x, w_gate):
    BT = 512
    grid = (T // BT,)
    return pl.pallas_call(
        _gate_body,
        grid=grid,
        in_specs=[
            pl.BlockSpec((BT, D), lambda i: (i, 0)),
            pl.BlockSpec((E, D), lambda i: (0, 0)),
        ],
        out_specs=[
            pl.BlockSpec((K, BT), lambda i: (0, i)),
            pl.BlockSpec((K, BT), lambda i: (0, i)),
        ],
        out_shape=[
            jax.ShapeDtypeStruct((K, T), jnp.float32),
            jax.ShapeDtypeStruct((K, T), jnp.int32),
        ],
    )(x, w_gate)


def _routing(e_top, w_top):
    eflat = e_top.reshape(S)
    wflat = w_top.reshape(S)
    order = jnp.argsort(eflat, stable=True)
    sorted_e = eflat[order]
    counts = jnp.bincount(eflat, length=E)
    cum = jnp.concatenate([jnp.zeros(1, jnp.int32),
                           jnp.cumsum(counts)]).astype(jnp.int32)
    nblk = (counts + B - 1) // B
    bcum = jnp.concatenate([jnp.zeros(1, jnp.int32),
                            jnp.cumsum(nblk)]).astype(jnp.int32)
    pad_off = bcum * B
    barange = jnp.arange(NB, dtype=jnp.int32)
    be = jnp.clip(jnp.searchsorted(bcum[1:], barange, side='right'),
                  0, E - 1).astype(jnp.int32)
    nvalid = jnp.clip(counts[be] - (barange - bcum[be]) * B, 0, B)
    j = jnp.arange(S, dtype=jnp.int32)
    padpos = pad_off[sorted_e] + (j - cum[sorted_e])
    tid_pad = jnp.zeros(S_PAD, jnp.int32).at[padpos].set(
        (order % T).astype(jnp.int32))
    ws_pad = jnp.zeros(S_PAD, jnp.float32).at[padpos].set(wflat[order])
    posflat = jnp.zeros(S, jnp.int32).at[order].set(padpos)
    pos0, pos1 = posflat[:T], posflat[T:]
    return be, nvalid.astype(jnp.int32), tid_pad, ws_pad, pos0, pos1


def _expert_body(be_ref, nv_ref, xs_ref, ws_ref, w1_ref, w2_ref, out_ref):
    b = pl.program_id(0)

    @pl.when(nv_ref[b] > 0)
    def _():
        y = lax.dot_general(xs_ref[...], w1_ref[0],
                            (((1,), (1,)), ((), ())),
                            preferred_element_type=jnp.float32)
        g = y[:, H:]
        h = (y[:, :H] * (g * jax.nn.sigmoid(g))).astype(jnp.bfloat16)
        z = lax.dot_general(h, w2_ref[0], (((1,), (1,)), ((), ())),
                            preferred_element_type=jnp.float32)
        out_ref[...] = z * ws_ref[...]


def _experts(xs_bf, ws_col, e_w1b, e_w2b, be, nvalid):
    grid_spec = pltpu.PrefetchScalarGridSpec(
        num_scalar_prefetch=2,
        grid=(NB,),
        in_specs=[
            pl.BlockSpec((B, D), lambda b, be, nv: (b, 0)),
            pl.BlockSpec((B, 1), lambda b, be, nv: (b, 0)),
            pl.BlockSpec((1, H2, D), lambda b, be, nv: (be[b], 0, 0)),
            pl.BlockSpec((1, D, H), lambda b, be, nv: (be[b], 0, 0)),
        ],
        out_specs=pl.BlockSpec((B, D), lambda b, be, nv: (b, 0)),
    )
    return pl.pallas_call(
        _expert_body,
        grid_spec=grid_spec,
        out_shape=jax.ShapeDtypeStruct((S_PAD, D), jnp.float32),
    )(be, nvalid, xs_bf, ws_col, e_w1b, e_w2b)


def _shared_body(x_ref, w1_ref, w2_ref, out_ref):
    y = lax.dot_general(x_ref[...], w1_ref[...], (((1,), (1,)), ((), ())),
                        preferred_element_type=jnp.float32)
    g = y[:, H:]
    h = (y[:, :H] * (g * jax.nn.sigmoid(g))).astype(jnp.bfloat16)
    out_ref[...] = lax.dot_general(h, w2_ref[...], (((1,), (1,)), ((), ())),
                                   preferred_element_type=jnp.float32)


def _shared(x_bf, s_w1b, s_w2b):
    BS = 256
    return pl.pallas_call(
        _shared_body,
        grid=(T // BS,),
        in_specs=[
            pl.BlockSpec((BS, D), lambda i: (i, 0)),
            pl.BlockSpec((H2, D), lambda i: (0, 0)),
            pl.BlockSpec((D, H), lambda i: (0, 0)),
        ],
        out_specs=pl.BlockSpec((BS, D), lambda i: (i, 0)),
        out_shape=jax.ShapeDtypeStruct((T, D), jnp.float32),
    )(x_bf, s_w1b, s_w2b)


def kernel(x, w_gate, e_w1, e_w2, s_w1, s_w2):
    w_top, e_top = _gate(x, w_gate)
    be, nvalid, tid_pad, ws_pad, pos0, pos1 = _routing(e_top, w_top)

    x_bf = x.astype(jnp.bfloat16)
    e_w1b = e_w1.astype(jnp.bfloat16)
    e_w2b = e_w2.astype(jnp.bfloat16)
    s_w1b = s_w1.astype(jnp.bfloat16)
    s_w2b = s_w2.astype(jnp.bfloat16)

    xs_bf = x_bf[tid_pad]
    slot_out = _experts(xs_bf, ws_pad[:, None], e_w1b, e_w2b, be, nvalid)
    shared_out = _shared(x_bf, s_w1b, s_w2b)
    out = shared_out + slot_out[pos0] + slot_out[pos1]
    return out
